# baseline (speedup 1.0000x reference)
import jax
import jax.numpy as jnp
from jax import lax
from jax.experimental import pallas as pl
from jax.experimental.pallas import tpu as pltpu

_VMEM_LIMIT = 64 * 1024 * 1024


def _proj_kernel(x_ref, w_ref, q_ref, k_ref, ve_ref):
    qkv = jnp.dot(x_ref[...], w_ref[...], preferred_element_type=jnp.float32)
    d = q_ref.shape[-1]
    q_ref[...] = qkv[:, :d].astype(jnp.bfloat16)
    k_ref[...] = qkv[:, d:2 * d].astype(jnp.bfloat16)
    ve_ref[...] = jnp.concatenate(
        [qkv[:, 2 * d:3 * d].astype(jnp.bfloat16),
         jnp.ones((qkv.shape[0], ve_ref.shape[-1] - d), jnp.bfloat16)],
        axis=-1)


def _attn_kernel(q_ref, k_ref, ve_ref, o_ref):
    d = o_ref.shape[-1]
    s = lax.dot_general(
        q_ref[...], k_ref[...], (((1,), (1,)), ((), ())),
        preferred_element_type=jnp.float32)
    m = jnp.max(s, axis=-1, keepdims=True)
    p = jnp.exp(s - m).astype(jnp.bfloat16)
    ol = jnp.dot(p, ve_ref[...], preferred_element_type=jnp.float32)
    l = ol[:, d:d + 1]
    o_ref[...] = (ol[:, :d] * pl.reciprocal(l, approx=False)).astype(o_ref.dtype)


def kernel(x, w_qkv):
    seq, d_in = x.shape
    d = w_qkv.shape[1] // 3
    out_dtype = x.dtype

    tp = 1024
    q, k, ve = pl.pallas_call(
        _proj_kernel,
        out_shape=(
            jax.ShapeDtypeStruct((seq, d), jnp.bfloat16),
            jax.ShapeDtypeStruct((seq, d), jnp.bfloat16),
            jax.ShapeDtypeStruct((seq, 2 * d), jnp.bfloat16),
        ),
        grid=(seq // tp,),
        in_specs=[
            pl.BlockSpec((tp, d_in), lambda i: (i, 0)),
            pl.BlockSpec((d_in, 3 * d), lambda i: (0, 0)),
        ],
        out_specs=(
            pl.BlockSpec((tp, d), lambda i: (i, 0)),
            pl.BlockSpec((tp, d), lambda i: (i, 0)),
            pl.BlockSpec((tp, 2 * d), lambda i: (i, 0)),
        ),
        compiler_params=pltpu.CompilerParams(
            dimension_semantics=("parallel",),
            vmem_limit_bytes=_VMEM_LIMIT),
    )(x, w_qkv)

    tq = 256
    out = pl.pallas_call(
        _attn_kernel,
        out_shape=jax.ShapeDtypeStruct((seq, d), out_dtype),
        grid=(seq // tq,),
        in_specs=[
            pl.BlockSpec((tq, d), lambda i: (i, 0)),
            pl.BlockSpec((seq, d), lambda i: (0, 0)),
            pl.BlockSpec((seq, 2 * d), lambda i: (0, 0)),
        ],
        out_specs=pl.BlockSpec((tq, d), lambda i: (i, 0)),
        compiler_params=pltpu.CompilerParams(
            dimension_semantics=("parallel",),
            vmem_limit_bytes=_VMEM_LIMIT),
    )(q, k, ve)

    return out


# speedup vs baseline: 1.2819x; 1.2819x over previous
import jax
import jax.numpy as jnp
from jax import lax
from jax.experimental import pallas as pl
from jax.experimental.pallas import tpu as pltpu

_VMEM_LIMIT = 64 * 1024 * 1024


def _proj_kernel(x_ref, w_ref, qkv_ref):
    qkv_ref[...] = jnp.dot(
        x_ref[...], w_ref[...], preferred_element_type=jnp.float32
    ).astype(qkv_ref.dtype)


def _attn_kernel(q_ref, k_ref, v_ref, o_ref):
    s = lax.dot_general(
        q_ref[...], k_ref[...], (((1,), (1,)), ((), ())),
        preferred_element_type=jnp.float32)
    m = jnp.max(s, axis=-1, keepdims=True)
    p = jnp.exp(s - m)
    l = jnp.sum(p, axis=-1, keepdims=True)
    o = jnp.dot(p.astype(jnp.bfloat16), v_ref[...],
                preferred_element_type=jnp.float32)
    o_ref[...] = (o * pl.reciprocal(l, approx=False)).astype(o_ref.dtype)


def kernel(x, w_qkv):
    seq, d_in = x.shape
    d = w_qkv.shape[1] // 3
    out_dtype = x.dtype

    tp = 1024
    qkv = pl.pallas_call(
        _proj_kernel,
        out_shape=jax.ShapeDtypeStruct((seq, 3 * d), jnp.bfloat16),
        grid=(seq // tp,),
        in_specs=[
            pl.BlockSpec((tp, d_in), lambda i: (i, 0)),
            pl.BlockSpec((d_in, 3 * d), lambda i: (0, 0)),
        ],
        out_specs=pl.BlockSpec((tp, 3 * d), lambda i: (i, 0)),
        compiler_params=pltpu.CompilerParams(
            dimension_semantics=("parallel",),
            vmem_limit_bytes=_VMEM_LIMIT),
    )(x, w_qkv)

    tq = 256
    out = pl.pallas_call(
        _attn_kernel,
        out_shape=jax.ShapeDtypeStruct((seq, d), out_dtype),
        grid=(seq // tq,),
        in_specs=[
            pl.BlockSpec((tq, d), lambda i: (i, 0)),
            pl.BlockSpec((seq, d), lambda i: (0, 1)),
            pl.BlockSpec((seq, d), lambda i: (0, 2)),
        ],
        out_specs=pl.BlockSpec((tq, d), lambda i: (i, 0)),
        compiler_params=pltpu.CompilerParams(
            dimension_semantics=("parallel",),
            vmem_limit_bytes=_VMEM_LIMIT),
    )(qkv, qkv, qkv)

    return out


# speedup vs baseline: 1.3392x; 1.0447x over previous
import jax
import jax.numpy as jnp
from jax import lax
from jax.experimental import pallas as pl
from jax.experimental.pallas import tpu as pltpu

_VMEM_LIMIT = 64 * 1024 * 1024


_LOG2E = 1.4426950408889634


def _proj_kernel(x_ref, w_ref, qkv_ref):
    qkv = jnp.dot(x_ref[...], w_ref[...], preferred_element_type=jnp.float32)
    d = qkv_ref.shape[-1] // 3
    qkv = qkv * jnp.concatenate(
        [jnp.full((1, d), _LOG2E, jnp.float32),
         jnp.ones((1, 2 * d), jnp.float32)], axis=-1)
    qkv_ref[...] = qkv.astype(qkv_ref.dtype)


def _attn_kernel(q_ref, k_ref, v_ref, o_ref):
    s = lax.dot_general(
        q_ref[...], k_ref[...], (((1,), (1,)), ((), ())),
        preferred_element_type=jnp.float32)
    m = jnp.max(s, axis=-1, keepdims=True)
    p = jnp.exp2(s - m)
    l = jnp.sum(p, axis=-1, keepdims=True)
    o = jnp.dot(p.astype(jnp.bfloat16), v_ref[...],
                preferred_element_type=jnp.float32)
    o_ref[...] = (o * pl.reciprocal(l, approx=False)).astype(o_ref.dtype)


def kernel(x, w_qkv):
    seq, d_in = x.shape
    d = w_qkv.shape[1] // 3
    out_dtype = x.dtype

    tp = 1024
    qkv = pl.pallas_call(
        _proj_kernel,
        out_shape=jax.ShapeDtypeStruct((seq, 3 * d), jnp.bfloat16),
        grid=(seq // tp,),
        in_specs=[
            pl.BlockSpec((tp, d_in), lambda i: (i, 0)),
            pl.BlockSpec((d_in, 3 * d), lambda i: (0, 0)),
        ],
        out_specs=pl.BlockSpec((tp, 3 * d), lambda i: (i, 0)),
        compiler_params=pltpu.CompilerParams(
            dimension_semantics=("parallel",),
            vmem_limit_bytes=_VMEM_LIMIT),
    )(x, w_qkv)

    tq = 256
    out = pl.pallas_call(
        _attn_kernel,
        out_shape=jax.ShapeDtypeStruct((seq, d), out_dtype),
        grid=(seq // tq,),
        in_specs=[
            pl.BlockSpec((tq, d), lambda i: (i, 0)),
            pl.BlockSpec((seq, d), lambda i: (0, 1)),
            pl.BlockSpec((seq, d), lambda i: (0, 2)),
        ],
        out_specs=pl.BlockSpec((tq, d), lambda i: (i, 0)),
        compiler_params=pltpu.CompilerParams(
            dimension_semantics=("parallel",),
            vmem_limit_bytes=_VMEM_LIMIT),
    )(qkv, qkv, qkv)

    return out


# speedup vs baseline: 1.3490x; 1.0073x over previous
import jax
import jax.numpy as jnp
from jax import lax
from jax.experimental import pallas as pl
from jax.experimental.pallas import tpu as pltpu

_VMEM_LIMIT = 64 * 1024 * 1024
_LOG2E = 1.4426950408889634
_N_CHUNKS = 2


def _attn_kernel(x_ref, w_ref, o_ref, k_sc, v_sc):
    i = pl.program_id(0)
    d = o_ref.shape[-1]
    tq = o_ref.shape[0]

    @pl.when(i == 0)
    def _project_kv():
        x_all = x_ref[...]
        k_sc[...] = jnp.dot(x_all, w_ref[:, d:2 * d],
                            preferred_element_type=jnp.float32
                            ).astype(jnp.bfloat16)
        v_sc[...] = jnp.dot(x_all, w_ref[:, 2 * d:3 * d],
                            preferred_element_type=jnp.float32
                            ).astype(jnp.bfloat16)

    q = jnp.dot(x_ref[pl.ds(i * tq, tq), :], w_ref[:, :d],
                preferred_element_type=jnp.float32).astype(jnp.bfloat16)

    c = k_sc.shape[0] // _N_CHUNKS
    ms, ls, os_ = [], [], []
    for j in range(_N_CHUNKS):
        s = lax.dot_general(
            q, k_sc[j * c:(j + 1) * c, :], (((1,), (1,)), ((), ())),
            preferred_element_type=jnp.float32)
        m = jnp.max(s, axis=-1, keepdims=True)
        p = jnp.exp2(s - m)
        ls.append(jnp.sum(p, axis=-1, keepdims=True))
        os_.append(jnp.dot(p.astype(jnp.bfloat16), v_sc[j * c:(j + 1) * c, :],
                           preferred_element_type=jnp.float32))
        ms.append(m)
    mg = ms[0]
    for m in ms[1:]:
        mg = jnp.maximum(mg, m)
    o = jnp.zeros_like(os_[0])
    l = jnp.zeros_like(ls[0])
    for j in range(_N_CHUNKS):
        a = jnp.exp2(ms[j] - mg)
        o = o + os_[j] * a
        l = l + ls[j] * a
    o_ref[...] = (o * pl.reciprocal(l, approx=False)).astype(o_ref.dtype)


def kernel(x, w_qkv):
    seq, d_in = x.shape
    d = w_qkv.shape[1] // 3
    out_dtype = x.dtype

    w_prep = w_qkv * jnp.concatenate(
        [jnp.full((1, d), _LOG2E, w_qkv.dtype),
         jnp.ones((1, 2 * d), w_qkv.dtype)], axis=-1)

    tq = 256
    out = pl.pallas_call(
        _attn_kernel,
        out_shape=jax.ShapeDtypeStruct((seq, d), out_dtype),
        grid=(seq // tq,),
        in_specs=[
            pl.BlockSpec((seq, d_in), lambda i: (0, 0)),
            pl.BlockSpec((d_in, 3 * d), lambda i: (0, 0)),
        ],
        out_specs=pl.BlockSpec((tq, d), lambda i: (i, 0)),
        scratch_shapes=[
            pltpu.VMEM((seq, d), jnp.bfloat16),
            pltpu.VMEM((seq, d), jnp.bfloat16),
        ],
        compiler_params=pltpu.CompilerParams(
            dimension_semantics=("arbitrary",),
            vmem_limit_bytes=_VMEM_LIMIT),
    )(x, w_prep)

    return out


# speedup vs baseline: 1.4663x; 1.0870x over previous
import jax
import jax.numpy as jnp
from jax import lax
from jax.experimental import pallas as pl
from jax.experimental.pallas import tpu as pltpu

_VMEM_LIMIT = 64 * 1024 * 1024
_LOG2E = 1.4426950408889634
_N_CHUNKS = 4


def _proj_kernel(x_ref, w_ref, qkv_ref):
    qkv = jnp.dot(x_ref[...], w_ref[...], preferred_element_type=jnp.float32)
    d = qkv_ref.shape[-1] // 3
    qkv = qkv * jnp.concatenate(
        [jnp.full((1, d), _LOG2E, jnp.float32),
         jnp.ones((1, 2 * d), jnp.float32)], axis=-1)
    qkv_ref[...] = qkv.astype(qkv_ref.dtype)


def _attn_kernel(q_ref, k_ref, v_ref, o_ref):
    q = q_ref[...]
    c = k_ref.shape[0] // _N_CHUNKS
    ms, ls, os_ = [], [], []
    for j in range(_N_CHUNKS):
        s = lax.dot_general(
            q, k_ref[j * c:(j + 1) * c, :], (((1,), (1,)), ((), ())),
            preferred_element_type=jnp.float32)
        m = jnp.max(s, axis=-1, keepdims=True)
        p = jnp.exp2(s - m)
        ls.append(jnp.sum(p, axis=-1, keepdims=True))
        os_.append(jnp.dot(p.astype(jnp.bfloat16), v_ref[j * c:(j + 1) * c, :],
                           preferred_element_type=jnp.float32))
        ms.append(m)
    mg = ms[0]
    for m in ms[1:]:
        mg = jnp.maximum(mg, m)
    o = jnp.zeros_like(os_[0])
    l = jnp.zeros_like(ls[0])
    for j in range(_N_CHUNKS):
        a = jnp.exp2(ms[j] - mg)
        o = o + os_[j] * a
        l = l + ls[j] * a
    o_ref[...] = (o * pl.reciprocal(l, approx=False)).astype(o_ref.dtype)


def kernel(x, w_qkv):
    seq, d_in = x.shape
    d = w_qkv.shape[1] // 3
    out_dtype = x.dtype

    tp = 1024
    qkv = pl.pallas_call(
        _proj_kernel,
        out_shape=jax.ShapeDtypeStruct((seq, 3 * d), jnp.bfloat16),
        grid=(seq // tp,),
        in_specs=[
            pl.BlockSpec((tp, d_in), lambda i: (i, 0)),
            pl.BlockSpec((d_in, 3 * d), lambda i: (0, 0)),
        ],
        out_specs=pl.BlockSpec((tp, 3 * d), lambda i: (i, 0)),
        compiler_params=pltpu.CompilerParams(
            dimension_semantics=("parallel",),
            vmem_limit_bytes=_VMEM_LIMIT),
    )(x, w_qkv)

    tq = 512
    out = pl.pallas_call(
        _attn_kernel,
        out_shape=jax.ShapeDtypeStruct((seq, d), out_dtype),
        grid=(seq // tq,),
        in_specs=[
            pl.BlockSpec((tq, d), lambda i: (i, 0)),
            pl.BlockSpec((seq, d), lambda i: (0, 1)),
            pl.BlockSpec((seq, d), lambda i: (0, 2)),
        ],
        out_specs=pl.BlockSpec((tq, d), lambda i: (i, 0)),
        compiler_params=pltpu.CompilerParams(
            dimension_semantics=("parallel",),
            vmem_limit_bytes=_VMEM_LIMIT),
    )(qkv, qkv, qkv)

    return out


# speedup vs baseline: 1.5880x; 1.0830x over previous
import jax
import jax.numpy as jnp
from jax import lax
from jax.experimental import pallas as pl
from jax.experimental.pallas import tpu as pltpu

_VMEM_LIMIT = 64 * 1024 * 1024
_LOG2E = 1.4426950408889634
_N_CHUNKS = 8


def _proj_kernel(x_ref, w_ref, qkv_ref):
    qkv = jnp.dot(x_ref[...], w_ref[...], preferred_element_type=jnp.float32)
    d = qkv_ref.shape[-1] // 3
    qkv = qkv * jnp.concatenate(
        [jnp.full((1, d), _LOG2E, jnp.float32),
         jnp.ones((1, 2 * d), jnp.float32)], axis=-1)
    qkv_ref[...] = qkv.astype(qkv_ref.dtype)


def _attn_kernel(q_ref, k_ref, v_ref, o_ref):
    q = q_ref[...]
    c = k_ref.shape[0] // _N_CHUNKS
    ms, ls, os_ = [], [], []
    for j in range(_N_CHUNKS):
        s = lax.dot_general(
            q, k_ref[j * c:(j + 1) * c, :], (((1,), (1,)), ((), ())),
            preferred_element_type=jnp.float32)
        m = jnp.max(s, axis=-1, keepdims=True)
        p = jnp.exp2(s - m)
        ls.append(jnp.sum(p, axis=-1, keepdims=True))
        os_.append(jnp.dot(p.astype(jnp.bfloat16), v_ref[j * c:(j + 1) * c, :],
                           preferred_element_type=jnp.float32))
        ms.append(m)
    mg = ms[0]
    for m in ms[1:]:
        mg = jnp.maximum(mg, m)
    o = jnp.zeros_like(os_[0])
    l = jnp.zeros_like(ls[0])
    for j in range(_N_CHUNKS):
        a = jnp.exp2(ms[j] - mg)
        o = o + os_[j] * a
        l = l + ls[j] * a
    o_ref[...] = (o * pl.reciprocal(l, approx=False)).astype(o_ref.dtype)


def kernel(x, w_qkv):
    seq, d_in = x.shape
    d = w_qkv.shape[1] // 3
    out_dtype = x.dtype

    tp = 1024
    qkv = pl.pallas_call(
        _proj_kernel,
        out_shape=jax.ShapeDtypeStruct((seq, 3 * d), jnp.bfloat16),
        grid=(seq // tp,),
        in_specs=[
            pl.BlockSpec((tp, d_in), lambda i: (i, 0)),
            pl.BlockSpec((d_in, 3 * d), lambda i: (0, 0)),
        ],
        out_specs=pl.BlockSpec((tp, 3 * d), lambda i: (i, 0)),
        compiler_params=pltpu.CompilerParams(
            dimension_semantics=("parallel",),
            vmem_limit_bytes=_VMEM_LIMIT),
    )(x, w_qkv)

    tq = 1024
    out = pl.pallas_call(
        _attn_kernel,
        out_shape=jax.ShapeDtypeStruct((seq, d), out_dtype),
        grid=(seq // tq,),
        in_specs=[
            pl.BlockSpec((tq, d), lambda i: (i, 0)),
            pl.BlockSpec((seq, d), lambda i: (0, 1)),
            pl.BlockSpec((seq, d), lambda i: (0, 2)),
        ],
        out_specs=pl.BlockSpec((tq, d), lambda i: (i, 0)),
        compiler_params=pltpu.CompilerParams(
            dimension_semantics=("parallel",),
            vmem_limit_bytes=_VMEM_LIMIT),
    )(qkv, qkv, qkv)

    return out


# speedup vs baseline: 1.6234x; 1.0223x over previous
import jax
import jax.numpy as jnp
from jax import lax
from jax.experimental import pallas as pl
from jax.experimental.pallas import tpu as pltpu

_VMEM_LIMIT = 64 * 1024 * 1024
_LOG2E = 1.4426950408889634
_N_CHUNKS = 8


def _attn_kernel(x_ref, w_ref, o_ref, k_sc, v_sc):
    i = pl.program_id(0)
    d = o_ref.shape[-1]
    tq = o_ref.shape[0]

    @pl.when(i == 0)
    def _project_kv():
        x_all = x_ref[...]
        k_sc[...] = jnp.dot(x_all, w_ref[:, d:2 * d],
                            preferred_element_type=jnp.float32
                            ).astype(jnp.bfloat16)
        v_sc[...] = jnp.dot(x_all, w_ref[:, 2 * d:3 * d],
                            preferred_element_type=jnp.float32
                            ).astype(jnp.bfloat16)

    q = (jnp.dot(x_ref[pl.ds(i * tq, tq), :], w_ref[:, :d],
                 preferred_element_type=jnp.float32)
         * _LOG2E).astype(jnp.bfloat16)

    c = k_sc.shape[0] // _N_CHUNKS
    ms, ls, os_ = [], [], []
    for j in range(_N_CHUNKS):
        s = lax.dot_general(
            q, k_sc[j * c:(j + 1) * c, :], (((1,), (1,)), ((), ())),
            preferred_element_type=jnp.float32)
        m = jnp.max(s, axis=-1, keepdims=True)
        p = jnp.exp2(s - m)
        ls.append(jnp.sum(p, axis=-1, keepdims=True))
        os_.append(jnp.dot(p.astype(jnp.bfloat16), v_sc[j * c:(j + 1) * c, :],
                           preferred_element_type=jnp.float32))
        ms.append(m)
    mg = ms[0]
    for m in ms[1:]:
        mg = jnp.maximum(mg, m)
    o = jnp.zeros_like(os_[0])
    l = jnp.zeros_like(ls[0])
    for j in range(_N_CHUNKS):
        a = jnp.exp2(ms[j] - mg)
        o = o + os_[j] * a
        l = l + ls[j] * a
    o_ref[...] = (o * pl.reciprocal(l, approx=False)).astype(o_ref.dtype)


def kernel(x, w_qkv):
    seq, d_in = x.shape
    d = w_qkv.shape[1] // 3
    out_dtype = x.dtype

    tq = 1024
    out = pl.pallas_call(
        _attn_kernel,
        out_shape=jax.ShapeDtypeStruct((seq, d), out_dtype),
        grid=(seq // tq,),
        in_specs=[
            pl.BlockSpec((seq, d_in), lambda i: (0, 0)),
            pl.BlockSpec((d_in, 3 * d), lambda i: (0, 0)),
        ],
        out_specs=pl.BlockSpec((tq, d), lambda i: (i, 0)),
        scratch_shapes=[
            pltpu.VMEM((seq, d), jnp.bfloat16),
            pltpu.VMEM((seq, d), jnp.bfloat16),
        ],
        compiler_params=pltpu.CompilerParams(
            dimension_semantics=("arbitrary",),
            vmem_limit_bytes=_VMEM_LIMIT),
    )(x, w_qkv)

    return out
